# baseline (speedup 1.0000x reference)
import dataclasses

import jax
import jax.numpy as jnp
from jax import lax
from jax.experimental import pallas as pl
from jax.experimental.pallas import tpu as pltpu
from jax.experimental.pallas import tpu_sc as plsc

ROWS = 8192
DIM = 4096

NC = 2
NS = 16
NW = NC * NS
R_PER_W = ROWS // NW
CH = 4
CHW = CH * DIM
NCH = R_PER_W // CH
NGRP = DIM // 16
UNROLL = 8


def _build_flatidx(perm_v, fidx):
    @pl.loop(0, NGRP)
    def _(j):
        pv = perm_v[pl.ds(j * 16, 16)]
        for r in range(CH):
            fidx[pl.ds(r * DIM + j * 16, 16)] = pv + r * DIM


def _shuffle(fidx, in_b, out_b):
    @pl.loop(0, CH * NGRP, step=UNROLL)
    def _(g):
        base = g * 16
        for u in range(UNROLL):
            off = base + u * 16
            idx = fidx[pl.ds(off, 16)]
            out_b[pl.ds(off, 16)] = plsc.load_gather(in_b, [idx])


def _sc_body(x_hbm, perm_hbm, o_hbm, perm_v, fidx, in0, in1, out0, out1,
             si0, si1, so0, so1):
    wid = lax.axis_index("s") * NC + lax.axis_index("c")
    base = wid * R_PER_W * DIM

    pltpu.sync_copy(perm_hbm, perm_v)
    _build_flatidx(perm_v, fidx)

    def elems(c):
        return pl.ds(base + c * CHW, CHW)

    pltpu.async_copy(x_hbm.at[elems(0)], in0, si0)

    @pl.loop(0, NCH, step=2)
    def _(c):
        pltpu.async_copy(x_hbm.at[elems(c + 1)], in1, si1)
        pltpu.make_async_copy(x_hbm.at[elems(c)], in0, si0).wait()

        @pl.when(c >= 2)
        def _():
            pltpu.make_async_copy(out0, o_hbm.at[elems(c - 2)], so0).wait()

        _shuffle(fidx, in0, out0)
        pltpu.async_copy(out0, o_hbm.at[elems(c)], so0)

        @pl.when(c + 2 < NCH)
        def _():
            pltpu.async_copy(x_hbm.at[elems(c + 2)], in0, si0)

        pltpu.make_async_copy(x_hbm.at[elems(c + 1)], in1, si1).wait()

        @pl.when(c >= 2)
        def _():
            pltpu.make_async_copy(out1, o_hbm.at[elems(c - 1)], so1).wait()

        _shuffle(fidx, in1, out1)
        pltpu.async_copy(out1, o_hbm.at[elems(c + 1)], so1)

    pltpu.make_async_copy(out0, o_hbm.at[elems(NCH - 2)], so0).wait()
    pltpu.make_async_copy(out1, o_hbm.at[elems(NCH - 1)], so1).wait()


def kernel(x, perm):
    mesh = plsc.VectorSubcoreMesh(core_axis_name="c", subcore_axis_name="s")
    cp = pltpu.CompilerParams()
    if "needs_layout_passes" in pltpu.CompilerParams.__dataclass_fields__:
        cp = dataclasses.replace(cp, needs_layout_passes=False)
    kfn = pl.kernel(
        _sc_body,
        mesh=mesh,
        compiler_params=cp,
        out_type=jax.ShapeDtypeStruct((ROWS * DIM,), jnp.float32),
        scratch_types=[
            pltpu.VMEM((DIM,), jnp.int32),
            pltpu.VMEM((CHW,), jnp.int32),
            pltpu.VMEM((CHW,), jnp.float32),
            pltpu.VMEM((CHW,), jnp.float32),
            pltpu.VMEM((CHW,), jnp.float32),
            pltpu.VMEM((CHW,), jnp.float32),
            pltpu.SemaphoreType.DMA,
            pltpu.SemaphoreType.DMA,
            pltpu.SemaphoreType.DMA,
            pltpu.SemaphoreType.DMA,
        ],
    )
    return kfn(x.reshape(ROWS * DIM), perm).reshape(ROWS, DIM)


# speedup vs baseline: 1.4911x; 1.4911x over previous
import dataclasses

import jax
import jax.numpy as jnp
from jax import lax
from jax.experimental import pallas as pl
from jax.experimental.pallas import tpu as pltpu
from jax.experimental.pallas import tpu_sc as plsc

ROWS = 8192
DIM = 4096

NC = 2
NS = 16
NW = NC * NS
R_PER_W = ROWS // NW
CH = 4
CHW = CH * DIM
NCH = R_PER_W // CH
NGRP = DIM // 16
UNROLL = 8


def _build_flatidx(perm_v, fidx):
    @pl.loop(0, NGRP)
    def _(j):
        pv = perm_v[pl.ds(j * 16, 16)]
        for r in range(CH):
            fidx[pl.ds(r * DIM + j * 16, 16)] = pv + r * DIM


def _shuffle(fidx, in_b, out_b):
    @pl.loop(0, CH * NGRP, step=UNROLL)
    def _(g):
        base = g * 16
        idxs = [fidx[pl.ds(base + u * 16, 16)] for u in range(UNROLL)]
        vals = [plsc.load_gather(in_b, [idxs[u]]) for u in range(UNROLL)]
        for u in range(UNROLL):
            out_b[pl.ds(base + u * 16, 16)] = vals[u]


def _sc_body(x_hbm, perm_hbm, o_hbm, perm_v, fidx, in0, in1, out0, out1,
             si0, si1, so0, so1):
    wid = lax.axis_index("s") * NC + lax.axis_index("c")
    base = wid * R_PER_W * DIM

    pltpu.sync_copy(perm_hbm, perm_v)
    _build_flatidx(perm_v, fidx)

    def elems(c):
        return pl.ds(base + c * CHW, CHW)

    pltpu.async_copy(x_hbm.at[elems(0)], in0, si0)

    @pl.loop(0, NCH, step=2)
    def _(c):
        pltpu.async_copy(x_hbm.at[elems(c + 1)], in1, si1)
        pltpu.make_async_copy(x_hbm.at[elems(c)], in0, si0).wait()

        @pl.when(c >= 2)
        def _():
            pltpu.make_async_copy(out0, o_hbm.at[elems(c - 2)], so0).wait()

        _shuffle(fidx, in0, out0)
        pltpu.async_copy(out0, o_hbm.at[elems(c)], so0)

        @pl.when(c + 2 < NCH)
        def _():
            pltpu.async_copy(x_hbm.at[elems(c + 2)], in0, si0)

        pltpu.make_async_copy(x_hbm.at[elems(c + 1)], in1, si1).wait()

        @pl.when(c >= 2)
        def _():
            pltpu.make_async_copy(out1, o_hbm.at[elems(c - 1)], so1).wait()

        _shuffle(fidx, in1, out1)
        pltpu.async_copy(out1, o_hbm.at[elems(c + 1)], so1)

    pltpu.make_async_copy(out0, o_hbm.at[elems(NCH - 2)], so0).wait()
    pltpu.make_async_copy(out1, o_hbm.at[elems(NCH - 1)], so1).wait()


def kernel(x, perm):
    mesh = plsc.VectorSubcoreMesh(core_axis_name="c", subcore_axis_name="s")
    cp = pltpu.CompilerParams()
    if "needs_layout_passes" in pltpu.CompilerParams.__dataclass_fields__:
        cp = dataclasses.replace(cp, needs_layout_passes=False)
    kfn = pl.kernel(
        _sc_body,
        mesh=mesh,
        compiler_params=cp,
        out_type=jax.ShapeDtypeStruct((ROWS * DIM,), jnp.float32),
        scratch_types=[
            pltpu.VMEM((DIM,), jnp.int32),
            pltpu.VMEM((CHW,), jnp.int32),
            pltpu.VMEM((CHW,), jnp.float32),
            pltpu.VMEM((CHW,), jnp.float32),
            pltpu.VMEM((CHW,), jnp.float32),
            pltpu.VMEM((CHW,), jnp.float32),
            pltpu.SemaphoreType.DMA,
            pltpu.SemaphoreType.DMA,
            pltpu.SemaphoreType.DMA,
            pltpu.SemaphoreType.DMA,
        ],
    )
    return kfn(x.reshape(ROWS * DIM), perm).reshape(ROWS, DIM)
